# baseline (speedup 1.0000x reference)
import functools

import jax
import jax.numpy as jnp
import numpy as np
from jax import lax
from jax.experimental import pallas as pl
from jax.experimental.pallas import tpu as pltpu
from jax.experimental.pallas import tpu_sc as plsc

D_MODEL = 256
SKU_B, CAT_B, URL_B, PRICE_B, TYPE_B = 65536, 4096, 65536, 128, 8
B, L = 4096, 50
BL = B * L

NUM_CORES = 2
NUM_SUBCORES = 16
NW = NUM_CORES * NUM_SUBCORES
TOK_PER_W = BL // NW
CHUNK = 64
NCHUNK = TOK_PER_W // CHUNK
LANES = 16


def _sc_gather_sum3(ids_all, emb_sku, emb_cat, emb_url):
  mesh = plsc.VectorSubcoreMesh(core_axis_name="c", subcore_axis_name="s")

  @functools.partial(
      pl.kernel,
      out_type=jax.ShapeDtypeStruct((BL, D_MODEL), jnp.float32),
      mesh=mesh,
      scratch_types=[
          pltpu.VMEM((3, TOK_PER_W), jnp.int32),
          pltpu.VMEM((3, CHUNK, D_MODEL), jnp.float32),
          pltpu.VMEM((3, CHUNK, D_MODEL), jnp.float32),
          pltpu.SemaphoreType.DMA,
          pltpu.SemaphoreType.DMA,
      ],
  )
  def k(ids_hbm, t_sku, t_cat, t_url, out_hbm,
        idx_v, rows_a, rows_b, sem_a, sem_b):
    wid = lax.axis_index("s") * NUM_CORES + lax.axis_index("c")
    base0 = wid * TOK_PER_W
    pltpu.sync_copy(ids_hbm.at[:, pl.ds(base0, TOK_PER_W)], idx_v)

    def issue(c, rows_v, sem):
      off = c * CHUNK
      pltpu.async_copy(t_sku.at[idx_v.at[0, pl.ds(off, CHUNK)]],
                       rows_v.at[0], sem)
      pltpu.async_copy(t_cat.at[idx_v.at[1, pl.ds(off, CHUNK)]],
                       rows_v.at[1], sem)
      pltpu.async_copy(t_url.at[idx_v.at[2, pl.ds(off, CHUNK)]],
                       rows_v.at[2], sem)

    def drain(rows_v, sem):
      for t in range(3):
        pltpu.make_async_copy(out_hbm.at[pl.ds(0, CHUNK)],
                              rows_v.at[t], sem).wait()

    def process(c, rows_v, sem):
      drain(rows_v, sem)

      def sum_tok(i, carry):
        for j in range(D_MODEL // LANES):
          s = (rows_v[0, i, pl.ds(j * LANES, LANES)] +
               rows_v[1, i, pl.ds(j * LANES, LANES)] +
               rows_v[2, i, pl.ds(j * LANES, LANES)])
          rows_v[0, i, pl.ds(j * LANES, LANES)] = s
        return carry

      lax.fori_loop(0, CHUNK, sum_tok, 0)
      pltpu.sync_copy(rows_v.at[0], out_hbm.at[pl.ds(base0 + c * CHUNK, CHUNK)])

    issue(0, rows_a, sem_a)

    def pair_body(i2, carry):
      c0 = i2 * 2

      @pl.when(c0 + 1 < NCHUNK)
      def _():
        issue(c0 + 1, rows_b, sem_b)

      process(c0, rows_a, sem_a)

      @pl.when(c0 + 2 < NCHUNK)
      def _():
        issue(c0 + 2, rows_a, sem_a)

      @pl.when(c0 + 1 < NCHUNK)
      def _():
        process(c0 + 1, rows_b, sem_b)

      return carry

    lax.fori_loop(0, (NCHUNK + 1) // 2, pair_body, 0)

  return k(ids_all, emb_sku, emb_cat, emb_url)


TBLK = 2048
NBLK = BL // TBLK


def _tc_body(acc_ref, q_ref, tid_ref, pid_ref, w_ref, ttab_ref, ptab_ref,
             b_ref, g_ref, bb_ref, o_ref):
  acc = acc_ref[...].reshape(TBLK, D_MODEL)
  x = acc + jnp.dot(q_ref[...], w_ref[...],
                    preferred_element_type=jnp.float32) + b_ref[...]
  tid = tid_ref[...].reshape(1, TBLK)
  pid = pid_ref[...].reshape(1, TBLK)
  oh_t = (lax.broadcasted_iota(jnp.int32, (TYPE_B, TBLK), 0) == tid
          ).astype(jnp.float32)
  oh_p = (lax.broadcasted_iota(jnp.int32, (PRICE_B, TBLK), 0) == pid
          ).astype(jnp.float32)
  x = x + lax.dot_general(oh_t, ttab_ref[...], (((0,), (0,)), ((), ())),
                          preferred_element_type=jnp.float32)
  x = x + lax.dot_general(oh_p, ptab_ref[...], (((0,), (0,)), ((), ())),
                          preferred_element_type=jnp.float32)
  x = 0.5 * x * (1.0 + lax.erf(x * np.float32(0.7071067811865476)))
  mu = jnp.mean(x, axis=-1, keepdims=True)
  d = x - mu
  var = jnp.mean(d * d, axis=-1, keepdims=True)
  o_ref[...] = d * lax.rsqrt(var + np.float32(1e-5)) * g_ref[...] + bb_ref[...]


def _tc_tail(acc, qv, tid, pid, W_q, ttab, ptab, b_q, ln_g, ln_b):
  return pl.pallas_call(
      _tc_body,
      grid=(NBLK,),
      in_specs=[
          pl.BlockSpec((TBLK * D_MODEL,), lambda i: (i,)),
          pl.BlockSpec((TBLK, 16), lambda i: (i, 0)),
          pl.BlockSpec((1, 1, TBLK), lambda i: (i, 0, 0)),
          pl.BlockSpec((1, 1, TBLK), lambda i: (i, 0, 0)),
          pl.BlockSpec((16, D_MODEL), lambda i: (0, 0)),
          pl.BlockSpec((TYPE_B, D_MODEL), lambda i: (0, 0)),
          pl.BlockSpec((PRICE_B, D_MODEL), lambda i: (0, 0)),
          pl.BlockSpec((1, D_MODEL), lambda i: (0, 0)),
          pl.BlockSpec((1, D_MODEL), lambda i: (0, 0)),
          pl.BlockSpec((1, D_MODEL), lambda i: (0, 0)),
      ],
      out_specs=pl.BlockSpec((TBLK, D_MODEL), lambda i: (i, 0)),
      out_shape=jax.ShapeDtypeStruct((BL, D_MODEL), jnp.float32),
  )(acc, qv, tid, pid, W_q, ttab, ptab, b_q, ln_g, ln_b)


def kernel(type_id, sku_id, cat_id, price_id, url_id, query_vec,
           emb_type, emb_sku, emb_cat, emb_url, emb_price,
           W_q, b_q, ln_g, ln_b):
  ids_all = jnp.stack([
      (sku_id.astype(jnp.int32) % SKU_B).reshape(-1),
      (cat_id.astype(jnp.int32) % CAT_B).reshape(-1),
      (url_id.astype(jnp.int32) % URL_B).reshape(-1),
  ])
  acc = _sc_gather_sum3(ids_all, emb_sku, emb_cat, emb_url).reshape(BL * D_MODEL)
  qv = query_vec.reshape(BL, 16)
  tid = (type_id.astype(jnp.int32) % TYPE_B).reshape(NBLK, 1, TBLK)
  pid = (price_id.astype(jnp.int32) % PRICE_B).reshape(NBLK, 1, TBLK)
  out = _tc_tail(acc, qv, tid, pid, W_q, emb_type, emb_price,
                 b_q.reshape(1, D_MODEL), ln_g.reshape(1, D_MODEL),
                 ln_b.reshape(1, D_MODEL))
  return out.reshape(B, L, D_MODEL)


# speedup vs baseline: 1.2468x; 1.2468x over previous
import functools

import jax
import jax.numpy as jnp
import numpy as np
from jax import lax
from jax.experimental import pallas as pl
from jax.experimental.pallas import tpu as pltpu
from jax.experimental.pallas import tpu_sc as plsc

D_MODEL = 256
SKU_B, CAT_B, URL_B, PRICE_B, TYPE_B = 65536, 4096, 65536, 128, 8
B, L = 4096, 50
BL = B * L

NUM_CORES = 2
NUM_SUBCORES = 16
NW = NUM_CORES * NUM_SUBCORES
TOK_PER_W = BL // NW
CHUNK = 64
NCHUNK = TOK_PER_W // CHUNK
LANES = 16


def _sc_gather_sum3(ids_all, emb_sku, emb_cat, emb_url):
  mesh = plsc.VectorSubcoreMesh(core_axis_name="c", subcore_axis_name="s")

  @functools.partial(
      pl.kernel,
      out_type=jax.ShapeDtypeStruct((BL, D_MODEL), jnp.float32),
      mesh=mesh,
      scratch_types=[
          pltpu.VMEM((3, TOK_PER_W), jnp.int32),
          pltpu.VMEM((3, CHUNK, D_MODEL), jnp.float32),
          pltpu.VMEM((3, CHUNK, D_MODEL), jnp.float32),
          pltpu.SemaphoreType.DMA,
          pltpu.SemaphoreType.DMA,
      ],
  )
  def k(ids_hbm, t_sku, t_cat, t_url, out_hbm,
        idx_v, rows_a, rows_b, sem_a, sem_b):
    wid = lax.axis_index("s") * NUM_CORES + lax.axis_index("c")
    base0 = wid * TOK_PER_W
    pltpu.sync_copy(ids_hbm.at[:, pl.ds(base0, TOK_PER_W)], idx_v)

    def issue(c, rows_v, sem):
      off = c * CHUNK
      pltpu.async_copy(t_sku.at[idx_v.at[0, pl.ds(off, CHUNK)]],
                       rows_v.at[0], sem)
      pltpu.async_copy(t_cat.at[idx_v.at[1, pl.ds(off, CHUNK)]],
                       rows_v.at[1], sem)
      pltpu.async_copy(t_url.at[idx_v.at[2, pl.ds(off, CHUNK)]],
                       rows_v.at[2], sem)

    def drain(rows_v, sem):
      for t in range(3):
        pltpu.make_async_copy(out_hbm.at[pl.ds(0, CHUNK)],
                              rows_v.at[t], sem).wait()

    def process(c, rows_v, sem):
      drain(rows_v, sem)

      def sum_tok(i, carry):
        for j in range(D_MODEL // LANES):
          s = (rows_v[0, i, pl.ds(j * LANES, LANES)] +
               rows_v[1, i, pl.ds(j * LANES, LANES)] +
               rows_v[2, i, pl.ds(j * LANES, LANES)])
          rows_v[0, i, pl.ds(j * LANES, LANES)] = s
        return carry

      lax.fori_loop(0, CHUNK, sum_tok, 0)
      pltpu.sync_copy(rows_v.at[0], out_hbm.at[pl.ds(base0 + c * CHUNK, CHUNK)])

    issue(0, rows_a, sem_a)

    def pair_body(i2, carry):
      c0 = i2 * 2

      @pl.when(c0 + 1 < NCHUNK)
      def _():
        issue(c0 + 1, rows_b, sem_b)

      process(c0, rows_a, sem_a)

      @pl.when(c0 + 2 < NCHUNK)
      def _():
        issue(c0 + 2, rows_a, sem_a)

      @pl.when(c0 + 1 < NCHUNK)
      def _():
        process(c0 + 1, rows_b, sem_b)

      return carry

    lax.fori_loop(0, (NCHUNK + 1) // 2, pair_body, 0)

  return k(ids_all, emb_sku, emb_cat, emb_url)


TBLK = 4096
NBLK = BL // TBLK


def _tc_body(acc_ref, q_ref, tid_ref, pid_ref, w_ref, ttab_ref, ptab_ref,
             b_ref, g_ref, bb_ref, o_ref):
  x = acc_ref[...] + jnp.dot(q_ref[...], w_ref[...],
                             preferred_element_type=jnp.float32) + b_ref[...]
  tid = tid_ref[...].reshape(1, TBLK)
  pid = pid_ref[...].reshape(1, TBLK)
  oh_t = (lax.broadcasted_iota(jnp.int32, (TYPE_B, TBLK), 0) == tid
          ).astype(jnp.bfloat16)
  oh_p = (lax.broadcasted_iota(jnp.int32, (PRICE_B, TBLK), 0) == pid
          ).astype(jnp.bfloat16)
  x = x + lax.dot_general(oh_t, ttab_ref[...].astype(jnp.bfloat16),
                          (((0,), (0,)), ((), ())),
                          preferred_element_type=jnp.float32)
  x = x + lax.dot_general(oh_p, ptab_ref[...].astype(jnp.bfloat16),
                          (((0,), (0,)), ((), ())),
                          preferred_element_type=jnp.float32)
  x = 0.5 * x * (1.0 + lax.erf(x * np.float32(0.7071067811865476)))
  mu = jnp.mean(x, axis=-1, keepdims=True)
  d = x - mu
  var = jnp.mean(d * d, axis=-1, keepdims=True)
  o_ref[...] = d * lax.rsqrt(var + np.float32(1e-5)) * g_ref[...] + bb_ref[...]


def _tc_tail(acc, qv, tid, pid, W_q, ttab, ptab, b_q, ln_g, ln_b):
  return pl.pallas_call(
      _tc_body,
      grid=(NBLK,),
      in_specs=[
          pl.BlockSpec((TBLK, D_MODEL), lambda i: (i, 0)),
          pl.BlockSpec((TBLK, 16), lambda i: (i, 0)),
          pl.BlockSpec((1, 1, TBLK), lambda i: (i, 0, 0)),
          pl.BlockSpec((1, 1, TBLK), lambda i: (i, 0, 0)),
          pl.BlockSpec((16, D_MODEL), lambda i: (0, 0)),
          pl.BlockSpec((TYPE_B, D_MODEL), lambda i: (0, 0)),
          pl.BlockSpec((PRICE_B, D_MODEL), lambda i: (0, 0)),
          pl.BlockSpec((1, D_MODEL), lambda i: (0, 0)),
          pl.BlockSpec((1, D_MODEL), lambda i: (0, 0)),
          pl.BlockSpec((1, D_MODEL), lambda i: (0, 0)),
      ],
      out_specs=pl.BlockSpec((TBLK, D_MODEL), lambda i: (i, 0)),
      out_shape=jax.ShapeDtypeStruct((BL, D_MODEL), jnp.float32),
  )(acc, qv, tid, pid, W_q, ttab, ptab, b_q, ln_g, ln_b)


def kernel(type_id, sku_id, cat_id, price_id, url_id, query_vec,
           emb_type, emb_sku, emb_cat, emb_url, emb_price,
           W_q, b_q, ln_g, ln_b):
  ids_all = jnp.stack([
      (sku_id.astype(jnp.int32) % SKU_B).reshape(-1),
      (cat_id.astype(jnp.int32) % CAT_B).reshape(-1),
      (url_id.astype(jnp.int32) % URL_B).reshape(-1),
  ])
  acc = _sc_gather_sum3(ids_all, emb_sku, emb_cat, emb_url)
  qv = query_vec.reshape(BL, 16)
  tid = (type_id.astype(jnp.int32) % TYPE_B).reshape(NBLK, 1, TBLK)
  pid = (price_id.astype(jnp.int32) % PRICE_B).reshape(NBLK, 1, TBLK)
  out = _tc_tail(acc, qv, tid, pid, W_q, emb_type, emb_price,
                 b_q.reshape(1, D_MODEL), ln_g.reshape(1, D_MODEL),
                 ln_b.reshape(1, D_MODEL))
  return out.reshape(B, L, D_MODEL)


# speedup vs baseline: 1.2553x; 1.0068x over previous
import functools

import jax
import jax.numpy as jnp
import numpy as np
from jax import lax
from jax.experimental import pallas as pl
from jax.experimental.pallas import tpu as pltpu
from jax.experimental.pallas import tpu_sc as plsc

D_MODEL = 256
SKU_B, CAT_B, URL_B, PRICE_B, TYPE_B = 65536, 4096, 65536, 128, 8
B, L = 4096, 50
BL = B * L

NUM_CORES = 2
NUM_SUBCORES = 16
NW = NUM_CORES * NUM_SUBCORES
TOK_PER_W = BL // NW
CHUNK = 64
NCHUNK = TOK_PER_W // CHUNK
LANES = 16


def _sc_gather_sum3(ids_all, emb_sku, emb_cat, emb_url):
  mesh = plsc.VectorSubcoreMesh(core_axis_name="c", subcore_axis_name="s")

  @functools.partial(
      pl.kernel,
      out_type=jax.ShapeDtypeStruct((BL, D_MODEL), jnp.float32),
      mesh=mesh,
      scratch_types=[
          pltpu.VMEM((3, TOK_PER_W), jnp.int32),
          pltpu.VMEM((3, CHUNK, D_MODEL), jnp.float32),
          pltpu.VMEM((3, CHUNK, D_MODEL), jnp.float32),
          pltpu.SemaphoreType.DMA,
          pltpu.SemaphoreType.DMA,
      ],
  )
  def k(ids_hbm, t_sku, t_cat, t_url, out_hbm,
        idx_v, rows_a, rows_b, sem_a, sem_b):
    wid = lax.axis_index("s") * NUM_CORES + lax.axis_index("c")
    base0 = wid * TOK_PER_W
    pltpu.sync_copy(ids_hbm.at[:, pl.ds(base0, TOK_PER_W)], idx_v)

    def issue(c, rows_v, sem):
      off = c * CHUNK
      pltpu.async_copy(t_sku.at[idx_v.at[0, pl.ds(off, CHUNK)]],
                       rows_v.at[0], sem)
      pltpu.async_copy(t_cat.at[idx_v.at[1, pl.ds(off, CHUNK)]],
                       rows_v.at[1], sem)
      pltpu.async_copy(t_url.at[idx_v.at[2, pl.ds(off, CHUNK)]],
                       rows_v.at[2], sem)

    def drain(rows_v, sem):
      for t in range(3):
        pltpu.make_async_copy(out_hbm.at[pl.ds(0, CHUNK)],
                              rows_v.at[t], sem).wait()

    def process(c, rows_v, sem):
      drain(rows_v, sem)

      @functools.partial(plsc.parallel_loop, 0, CHUNK, unroll=2)
      def sum_tok(i):
        for j in range(D_MODEL // LANES):
          s = (rows_v[0, i, pl.ds(j * LANES, LANES)] +
               rows_v[1, i, pl.ds(j * LANES, LANES)] +
               rows_v[2, i, pl.ds(j * LANES, LANES)])
          rows_v[0, i, pl.ds(j * LANES, LANES)] = s
      pltpu.sync_copy(rows_v.at[0], out_hbm.at[pl.ds(base0 + c * CHUNK, CHUNK)])

    issue(0, rows_a, sem_a)

    def pair_body(i2, carry):
      c0 = i2 * 2

      @pl.when(c0 + 1 < NCHUNK)
      def _():
        issue(c0 + 1, rows_b, sem_b)

      process(c0, rows_a, sem_a)

      @pl.when(c0 + 2 < NCHUNK)
      def _():
        issue(c0 + 2, rows_a, sem_a)

      @pl.when(c0 + 1 < NCHUNK)
      def _():
        process(c0 + 1, rows_b, sem_b)

      return carry

    lax.fori_loop(0, (NCHUNK + 1) // 2, pair_body, 0)

  return k(ids_all, emb_sku, emb_cat, emb_url)


TBLK = 4096
NBLK = BL // TBLK


def _tc_body(acc_ref, q_ref, tid_ref, pid_ref, w_ref, ttab_ref, ptab_ref,
             b_ref, g_ref, bb_ref, o_ref):
  x = acc_ref[...] + jnp.dot(q_ref[...], w_ref[...],
                             preferred_element_type=jnp.float32) + b_ref[...]
  tid = tid_ref[...].reshape(1, TBLK)
  pid = pid_ref[...].reshape(1, TBLK)
  oh_t = (lax.broadcasted_iota(jnp.int32, (TYPE_B, TBLK), 0) == tid
          ).astype(jnp.bfloat16)
  oh_p = (lax.broadcasted_iota(jnp.int32, (PRICE_B, TBLK), 0) == pid
          ).astype(jnp.bfloat16)
  x = x + lax.dot_general(oh_t, ttab_ref[...].astype(jnp.bfloat16),
                          (((0,), (0,)), ((), ())),
                          preferred_element_type=jnp.float32)
  x = x + lax.dot_general(oh_p, ptab_ref[...].astype(jnp.bfloat16),
                          (((0,), (0,)), ((), ())),
                          preferred_element_type=jnp.float32)
  x = 0.5 * x * (1.0 + lax.erf(x * np.float32(0.7071067811865476)))
  mu = jnp.mean(x, axis=-1, keepdims=True)
  d = x - mu
  var = jnp.mean(d * d, axis=-1, keepdims=True)
  o_ref[...] = d * lax.rsqrt(var + np.float32(1e-5)) * g_ref[...] + bb_ref[...]


def _tc_tail(acc, qv, tid, pid, W_q, ttab, ptab, b_q, ln_g, ln_b):
  return pl.pallas_call(
      _tc_body,
      grid=(NBLK,),
      in_specs=[
          pl.BlockSpec((TBLK, D_MODEL), lambda i: (i, 0)),
          pl.BlockSpec((TBLK, 16), lambda i: (i, 0)),
          pl.BlockSpec((1, 1, TBLK), lambda i: (i, 0, 0)),
          pl.BlockSpec((1, 1, TBLK), lambda i: (i, 0, 0)),
          pl.BlockSpec((16, D_MODEL), lambda i: (0, 0)),
          pl.BlockSpec((TYPE_B, D_MODEL), lambda i: (0, 0)),
          pl.BlockSpec((PRICE_B, D_MODEL), lambda i: (0, 0)),
          pl.BlockSpec((1, D_MODEL), lambda i: (0, 0)),
          pl.BlockSpec((1, D_MODEL), lambda i: (0, 0)),
          pl.BlockSpec((1, D_MODEL), lambda i: (0, 0)),
      ],
      out_specs=pl.BlockSpec((TBLK, D_MODEL), lambda i: (i, 0)),
      out_shape=jax.ShapeDtypeStruct((BL, D_MODEL), jnp.float32),
  )(acc, qv, tid, pid, W_q, ttab, ptab, b_q, ln_g, ln_b)


def kernel(type_id, sku_id, cat_id, price_id, url_id, query_vec,
           emb_type, emb_sku, emb_cat, emb_url, emb_price,
           W_q, b_q, ln_g, ln_b):
  ids_all = jnp.stack([
      (sku_id.astype(jnp.int32) % SKU_B).reshape(-1),
      (cat_id.astype(jnp.int32) % CAT_B).reshape(-1),
      (url_id.astype(jnp.int32) % URL_B).reshape(-1),
  ])
  acc = _sc_gather_sum3(ids_all, emb_sku, emb_cat, emb_url)
  qv = query_vec.reshape(BL, 16)
  tid = (type_id.astype(jnp.int32) % TYPE_B).reshape(NBLK, 1, TBLK)
  pid = (price_id.astype(jnp.int32) % PRICE_B).reshape(NBLK, 1, TBLK)
  out = _tc_tail(acc, qv, tid, pid, W_q, emb_type, emb_price,
                 b_q.reshape(1, D_MODEL), ln_g.reshape(1, D_MODEL),
                 ln_b.reshape(1, D_MODEL))
  return out.reshape(B, L, D_MODEL)
